# Initial kernel scaffold; baseline (speedup 1.0000x reference)
import functools

import jax
import jax.numpy as jnp
from jax import lax
from jax.experimental import pallas as pl
from jax.experimental.pallas import tpu as pltpu
from jax.experimental.pallas import tpu_sc as plsc

_N = 10000
_E = 320000
_IN = 128
_HID = 64
_NG = 16
_NC = 2
_NS = 16
_NP = 10240
_RPT = _NP // (_NC * _NS)
_EPT = _E // (_NC * _NS)
_EB = 80

_mesh = plsc.VectorSubcoreMesh(core_axis_name="c", subcore_axis_name="s")


@functools.partial(
    pl.kernel,
    out_type=jax.ShapeDtypeStruct((_NC, _NP), jnp.float32),
    mesh=_mesh,
    scratch_types=[
        pltpu.VMEM((_EB,), jnp.int32),
        pltpu.VMEM((_EB,), jnp.float32),
        pltpu.VMEM((_RPT,), jnp.float32),
        pltpu.VMEM_SHARED((_NP,), jnp.float32),
        pltpu.SemaphoreType.DMA,
    ],
)
def _sc_degree(dst_hbm, out_hbm, idx_v, ones_v, zero_v, acc_sh, sem):
    c = lax.axis_index("c")
    s = lax.axis_index("s")

    @pl.loop(0, _EB, step=16)
    def _(i):
        ones_v[pl.ds(i, 16)] = jnp.ones((16,), jnp.float32)

    @pl.loop(0, _RPT, step=16)
    def _(i):
        zero_v[pl.ds(i, 16)] = jnp.zeros((16,), jnp.float32)

    pltpu.sync_copy(zero_v, acc_sh.at[pl.ds(s * _RPT, _RPT)])
    plsc.subcore_barrier()

    base = (c * _NS + s) * _EPT

    @pl.loop(0, _EPT, step=_EB)
    def _(i):
        pltpu.sync_copy(dst_hbm.at[pl.ds(base + i, _EB)], idx_v)
        pltpu.sync_copy(ones_v, acc_sh.at[idx_v], add=True)

    plsc.subcore_barrier()
    pltpu.sync_copy(acc_sh.at[pl.ds(s * _RPT, _RPT)],
                    out_hbm.at[c, pl.ds(s * _RPT, _RPT)])


@functools.partial(
    pl.kernel,
    out_type=jax.ShapeDtypeStruct((_NC, _NP, _HID), jnp.float32),
    mesh=_mesh,
    scratch_types=[
        pltpu.VMEM((_EB,), jnp.int32),
        pltpu.VMEM((_EB,), jnp.int32),
        pltpu.VMEM((_EB, _HID), jnp.float32),
        pltpu.VMEM_SHARED((_NP, _HID), jnp.float32),
        pltpu.SemaphoreType.DMA,
    ],
)
def _sc_aggregate(table_hbm, src_hbm, dst_hbm, out_hbm,
                  isrc_v, idst_v, rows_v, acc_sh, sem):
    c = lax.axis_index("c")
    s = lax.axis_index("s")

    @pl.loop(0, _EB)
    def _(r):
        @pl.loop(0, _HID, step=16)
        def _(j):
            rows_v[r, pl.ds(j, 16)] = jnp.zeros((16,), jnp.float32)

    @pl.loop(0, _RPT, step=_EB)
    def _(r):
        pltpu.sync_copy(rows_v, acc_sh.at[pl.ds(s * _RPT + r, _EB)])

    plsc.subcore_barrier()

    base = (c * _NS + s) * _EPT

    @pl.loop(0, _EPT, step=_EB)
    def _(i):
        pltpu.sync_copy(src_hbm.at[pl.ds(base + i, _EB)], isrc_v)
        pltpu.sync_copy(dst_hbm.at[pl.ds(base + i, _EB)], idst_v)
        pltpu.async_copy(table_hbm.at[isrc_v], rows_v, sem).wait()
        pltpu.sync_copy(rows_v, acc_sh.at[idst_v], add=True)

    plsc.subcore_barrier()
    pltpu.sync_copy(acc_sh.at[pl.ds(s * _RPT, _RPT)],
                    out_hbm.at[c, pl.ds(s * _RPT, _RPT)])


def _dot(a, b):
    return jnp.dot(a, b, preferred_element_type=jnp.float32,
                   precision=lax.Precision.HIGHEST)


def _dinv_of(degp_ref):
    deg = degp_ref[0, :_N] + degp_ref[1, :_N] + 1.0
    return lax.rsqrt(deg)[:, None]


def _tc_matmul1_body(x_ref, w_ref, o_ref):
    o_ref[...] = _dot(x_ref[...], w_ref[...])


_tc_matmul1 = pl.pallas_call(
    _tc_matmul1_body,
    out_shape=jax.ShapeDtypeStruct((_N, _HID), jnp.float32),
)


def _tc_scale_body(h_ref, degp_ref, o_ref):
    o_ref[...] = _dinv_of(degp_ref) * h_ref[...]


_tc_scale = pl.pallas_call(
    _tc_scale_body,
    out_shape=jax.ShapeDtypeStruct((_N, _HID), jnp.float32),
)


def _tc_mid_body(p_ref, g_ref, degp_ref, b_ref, w_ref, o_ref):
    dinv = _dinv_of(degp_ref)
    h = p_ref[0, :_N, :] + p_ref[1, :_N, :] + g_ref[...]
    h = jnp.maximum(dinv * h + b_ref[...][None, :], 0.0)
    o_ref[...] = dinv * _dot(h, w_ref[...])


_tc_mid = pl.pallas_call(
    _tc_mid_body,
    out_shape=jax.ShapeDtypeStruct((_N, _HID), jnp.float32),
)


def _tc_final_body(p_ref, g_ref, degp_ref, b_ref, batch_ref, lw_ref, lb_ref,
                   o_ref):
    dinv = _dinv_of(degp_ref)
    h = p_ref[0, :_N, :] + p_ref[1, :_N, :] + g_ref[...]
    h = jnp.maximum(dinv * h + b_ref[...][None, :], 0.0)
    labels = lax.broadcasted_iota(jnp.int32, (1, _NG), 1)
    onehot = (batch_ref[...] == labels).astype(jnp.float32)
    sums = lax.dot_general(onehot, h, (((0,), (0,)), ((), ())),
                           preferred_element_type=jnp.float32,
                           precision=lax.Precision.HIGHEST)
    counts = jnp.sum(onehot, axis=0)[:, None]
    pooled = sums / jnp.maximum(counts, 1.0)
    o_ref[...] = _dot(pooled, lw_ref[...]) + lb_ref[...][None, :]


_tc_final = pl.pallas_call(
    _tc_final_body,
    out_shape=jax.ShapeDtypeStruct((_NG, 2), jnp.float32),
)


def kernel(x, edge_index, batch, W1, b1, W2, b2, W3, b3, lin_W, lin_b):
    src = edge_index[0].astype(jnp.int32)
    dst = edge_index[1].astype(jnp.int32)
    batch2 = batch.astype(jnp.int32).reshape(_N, 1)

    degp = _sc_degree(dst)
    h1 = _tc_matmul1(x, W1)
    g1 = _tc_scale(h1, degp)
    p1 = _sc_aggregate(g1, src, dst)
    g2 = _tc_mid(p1, g1, degp, b1, W2)
    p2 = _sc_aggregate(g2, src, dst)
    g3 = _tc_mid(p2, g2, degp, b2, W3)
    p3 = _sc_aggregate(g3, src, dst)
    return _tc_final(p3, g3, degp, b3, batch2, lin_W, lin_b)


# speedup vs baseline: 13.1769x; 13.1769x over previous
import functools

import jax
import jax.numpy as jnp
from jax import lax
from jax.experimental import pallas as pl
from jax.experimental.pallas import tpu as pltpu
from jax.experimental.pallas import tpu_sc as plsc

_N = 10000
_E = 320000
_IN = 128
_HID = 64
_NG = 16
_NC = 2
_NS = 16
_NP = 10240
_RPT = _NP // _NS
_EPT = _E // (_NC * _NS)
_EB = 80

_mesh = plsc.VectorSubcoreMesh(core_axis_name="c", subcore_axis_name="s")


@functools.partial(
    pl.kernel,
    out_type=jax.ShapeDtypeStruct((_NC * _NP,), jnp.float32),
    mesh=_mesh,
    scratch_types=[
        pltpu.VMEM((_EB,), jnp.int32),
        pltpu.VMEM((_EB,), jnp.float32),
        pltpu.VMEM((_RPT,), jnp.float32),
        pltpu.VMEM_SHARED((_NP,), jnp.float32),
        pltpu.SemaphoreType.DMA,
    ],
)
def _sc_degree(dst_hbm, out_hbm, idx_v, ones_v, zero_v, acc_sh, sem):
    c = lax.axis_index("c")
    s = lax.axis_index("s")

    @pl.loop(0, _EB, step=16)
    def _(i):
        ones_v[pl.ds(i, 16)] = jnp.ones((16,), jnp.float32)

    @pl.loop(0, _RPT, step=16)
    def _(i):
        zero_v[pl.ds(i, 16)] = jnp.zeros((16,), jnp.float32)

    pltpu.sync_copy(zero_v, acc_sh.at[pl.ds(s * _RPT, _RPT)])
    plsc.subcore_barrier()

    base = (c * _NS + s) * _EPT

    @pl.loop(0, _EPT, step=_EB)
    def _(i):
        pltpu.sync_copy(dst_hbm.at[pl.ds(base + i, _EB)], idx_v)
        pltpu.sync_copy(ones_v, acc_sh.at[idx_v], add=True)

    plsc.subcore_barrier()
    pltpu.sync_copy(acc_sh.at[pl.ds(s * _RPT, _RPT)], zero_v)
    pltpu.sync_copy(zero_v, out_hbm.at[pl.ds(c * _NP + s * _RPT, _RPT)])


@functools.partial(
    pl.kernel,
    out_type=jax.ShapeDtypeStruct((_NC, _NP, _HID), jnp.float32),
    mesh=_mesh,
    compiler_params=pltpu.CompilerParams(use_tc_tiling_on_sc=False),
    scratch_types=[
        pltpu.VMEM((_EB,), jnp.int32),
        pltpu.VMEM((_EB,), jnp.int32),
        pltpu.VMEM((_EB, _HID), jnp.float32),
        pltpu.VMEM_SHARED((_NP, _HID), jnp.float32),
        pltpu.SemaphoreType.DMA,
    ],
)
def _sc_aggregate(table_hbm, src_hbm, dst_hbm, out_hbm,
                  isrc_v, idst_v, rows_v, acc_sh, sem):
    c = lax.axis_index("c")
    s = lax.axis_index("s")

    @pl.loop(0, _EB)
    def _(r):
        @pl.loop(0, _HID, step=16)
        def _(j):
            rows_v[r, pl.ds(j, 16)] = jnp.zeros((16,), jnp.float32)

    @pl.loop(0, _RPT, step=_EB)
    def _(r):
        pltpu.sync_copy(rows_v, acc_sh.at[pl.ds(s * _RPT + r, _EB)])

    plsc.subcore_barrier()

    base = (c * _NS + s) * _EPT

    @pl.loop(0, _EPT, step=_EB)
    def _(i):
        pltpu.sync_copy(src_hbm.at[pl.ds(base + i, _EB)], isrc_v)
        pltpu.sync_copy(dst_hbm.at[pl.ds(base + i, _EB)], idst_v)
        pltpu.async_copy(table_hbm.at[isrc_v], rows_v, sem).wait()
        pltpu.sync_copy(rows_v, acc_sh.at[idst_v], add=True)

    plsc.subcore_barrier()
    @pl.loop(0, _RPT, step=_EB)
    def _(r):
        pltpu.sync_copy(acc_sh.at[pl.ds(s * _RPT + r, _EB)], rows_v)
        pltpu.sync_copy(rows_v, out_hbm.at[c, pl.ds(s * _RPT + r, _EB)])


def _dot(a, b):
    return jnp.dot(a, b, preferred_element_type=jnp.float32,
                   precision=lax.Precision.HIGHEST)


def _dinv_of(degp_ref):
    deg = degp_ref[:_N] + degp_ref[_NP:_NP + _N] + 1.0
    return lax.rsqrt(deg)[:, None]


def _tc_matmul1_body(x_ref, w_ref, o_ref):
    o_ref[...] = _dot(x_ref[...], w_ref[...])


_tc_matmul1 = pl.pallas_call(
    _tc_matmul1_body,
    out_shape=jax.ShapeDtypeStruct((_N, _HID), jnp.float32),
)


def _tc_scale_body(h_ref, degp_ref, o_ref):
    o_ref[...] = _dinv_of(degp_ref) * h_ref[...]


_tc_scale = pl.pallas_call(
    _tc_scale_body,
    out_shape=jax.ShapeDtypeStruct((_N, _HID), jnp.float32),
)


def _tc_mid_body(p_ref, g_ref, degp_ref, b_ref, w_ref, o_ref):
    dinv = _dinv_of(degp_ref)
    h = p_ref[0, :_N, :] + p_ref[1, :_N, :] + g_ref[...]
    h = jnp.maximum(dinv * h + b_ref[...][None, :], 0.0)
    o_ref[...] = dinv * _dot(h, w_ref[...])


_tc_mid = pl.pallas_call(
    _tc_mid_body,
    out_shape=jax.ShapeDtypeStruct((_N, _HID), jnp.float32),
)


def _tc_final_body(p_ref, g_ref, degp_ref, b_ref, batch_ref, lw_ref, lb_ref,
                   o_ref):
    dinv = _dinv_of(degp_ref)
    h = p_ref[0, :_N, :] + p_ref[1, :_N, :] + g_ref[...]
    h = jnp.maximum(dinv * h + b_ref[...][None, :], 0.0)
    labels = lax.broadcasted_iota(jnp.int32, (1, _NG), 1)
    onehot = (batch_ref[...] == labels).astype(jnp.float32)
    sums = lax.dot_general(onehot, h, (((0,), (0,)), ((), ())),
                           preferred_element_type=jnp.float32,
                           precision=lax.Precision.HIGHEST)
    counts = jnp.sum(onehot, axis=0)[:, None]
    pooled = sums / jnp.maximum(counts, 1.0)
    o_ref[...] = _dot(pooled, lw_ref[...]) + lb_ref[...][None, :]


_tc_final = pl.pallas_call(
    _tc_final_body,
    out_shape=jax.ShapeDtypeStruct((_NG, 2), jnp.float32),
)


def kernel(x, edge_index, batch, W1, b1, W2, b2, W3, b3, lin_W, lin_b):
    src = edge_index[0].astype(jnp.int32)
    dst = edge_index[1].astype(jnp.int32)
    batch2 = batch.astype(jnp.int32).reshape(_N, 1)

    degp = _sc_degree(dst)
    h1 = _tc_matmul1(x, W1)
    g1 = _tc_scale(h1, degp)
    p1 = _sc_aggregate(g1, src, dst)
    g2 = _tc_mid(p1, g1, degp, b1, W2)
    p2 = _sc_aggregate(g2, src, dst)
    g3 = _tc_mid(p2, g2, degp, b2, W3)
    p3 = _sc_aggregate(g3, src, dst)
    return _tc_final(p3, g3, degp, b3, batch2, lin_W, lin_b)


# speedup vs baseline: 40.8009x; 3.0964x over previous
import functools

import jax
import jax.numpy as jnp
from jax import lax
from jax.experimental import pallas as pl
from jax.experimental.pallas import tpu as pltpu
from jax.experimental.pallas import tpu_sc as plsc

_N = 10000
_E = 320000
_HID = 64
_NG = 16
_NC = 2
_NS = 16
_NP = 10240
_RPT = _NP // _NS
_BLK = 128
_EPAD = 327680
_NBT = _EPAD // (_NC * _NS * _BLK)
_RING = 8

_mesh = plsc.VectorSubcoreMesh(core_axis_name="c", subcore_axis_name="s")


@functools.partial(
    pl.kernel,
    out_type=jax.ShapeDtypeStruct((_NC * _NP,), jnp.float32),
    mesh=_mesh,
    scratch_types=[
        pltpu.VMEM((_NBT, _BLK), jnp.int32),
        pltpu.VMEM((_BLK,), jnp.float32),
        pltpu.VMEM((_RPT,), jnp.float32),
        pltpu.VMEM_SHARED((_NP,), jnp.float32),
        pltpu.SemaphoreType.DMA,
    ],
)
def _sc_degree(dst_hbm, out_hbm, idx_v, ones_v, zero_v, acc_sh, sem):
    c = lax.axis_index("c")
    s = lax.axis_index("s")

    @pl.loop(0, _BLK, step=16)
    def _(i):
        ones_v[pl.ds(i, 16)] = jnp.ones((16,), jnp.float32)

    @pl.loop(0, _RPT, step=16)
    def _(i):
        zero_v[pl.ds(i, 16)] = jnp.zeros((16,), jnp.float32)

    pltpu.sync_copy(zero_v, acc_sh.at[pl.ds(s * _RPT, _RPT)])
    row0 = (c * _NS + s) * _NBT
    pltpu.sync_copy(dst_hbm.at[pl.ds(row0, _NBT)], idx_v)
    plsc.subcore_barrier()

    @pl.loop(0, _NBT, step=20)
    def _(k):
        for j in range(20):
            pltpu.async_copy(ones_v, acc_sh.at[idx_v.at[k + j]], sem,
                             add=True)
        for j in range(20):
            pltpu.make_async_copy(ones_v, acc_sh.at[idx_v.at[k + j]],
                                  sem).wait()

    plsc.subcore_barrier()
    pltpu.sync_copy(acc_sh.at[pl.ds(s * _RPT, _RPT)], zero_v)
    pltpu.sync_copy(zero_v, out_hbm.at[pl.ds(c * _NP + s * _RPT, _RPT)])


@functools.partial(
    pl.kernel,
    out_type=jax.ShapeDtypeStruct((_NC, _NP, _HID), jnp.float32),
    mesh=_mesh,
    compiler_params=pltpu.CompilerParams(use_tc_tiling_on_sc=False),
    scratch_types=[
        pltpu.VMEM((_NBT, _BLK), jnp.int32),
        pltpu.VMEM((_NBT, _BLK), jnp.int32),
        pltpu.VMEM((_RING, _BLK, _HID), jnp.float32),
        pltpu.VMEM_SHARED((_NP, _HID), jnp.float32),
        pltpu.SemaphoreType.DMA,
    ] + [pltpu.SemaphoreType.DMA] * (2 * _RING),
)
def _sc_aggregate(table_hbm, src_hbm, dst_hbm, out_hbm,
                  isrc_v, idst_v, rows_v, acc_sh, sem, *slot_sems):
    c = lax.axis_index("c")
    s = lax.axis_index("s")
    sg = slot_sems[:_RING]
    ss = slot_sems[_RING:]

    def fire_gather(blk, b):
        pltpu.async_copy(table_hbm.at[isrc_v.at[blk]], rows_v.at[b], sg[b])

    def wait_gather(blk, b):
        pltpu.make_async_copy(table_hbm.at[isrc_v.at[blk]], rows_v.at[b],
                              sg[b]).wait()

    def fire_scatter(blk, b):
        pltpu.async_copy(rows_v.at[b], acc_sh.at[idst_v.at[blk]], ss[b],
                         add=True)

    def wait_scatter(blk, b):
        pltpu.make_async_copy(rows_v.at[b], acc_sh.at[idst_v.at[blk]],
                              ss[b]).wait()

    @pl.loop(0, _BLK)
    def _(r):
        @pl.loop(0, _HID, step=16)
        def _(j):
            rows_v[0, r, pl.ds(j, 16)] = jnp.zeros((16,), jnp.float32)

    @pl.loop(0, _RPT, step=_BLK)
    def _(r):
        pltpu.sync_copy(rows_v.at[0], acc_sh.at[pl.ds(s * _RPT + r, _BLK)])

    row0 = (c * _NS + s) * _NBT
    pltpu.sync_copy(src_hbm.at[pl.ds(row0, _NBT)], isrc_v)
    pltpu.sync_copy(dst_hbm.at[pl.ds(row0, _NBT)], idst_v)
    plsc.subcore_barrier()

    for b in range(_RING):
        fire_gather(b, b)
    for b in range(_RING):
        wait_gather(b, b)
        fire_scatter(b, b)

    @pl.loop(_RING, _NBT, step=_RING)
    def _(g):
        for b in range(_RING):
            wait_scatter(g - _RING + b, b)
            fire_gather(g + b, b)
        for b in range(_RING):
            wait_gather(g + b, b)
            fire_scatter(g + b, b)

    for b in range(_RING):
        wait_scatter(_NBT - _RING + b, b)

    plsc.subcore_barrier()
    @pl.loop(0, _RPT, step=_BLK)
    def _(r):
        pltpu.sync_copy(acc_sh.at[pl.ds(s * _RPT + r, _BLK)], rows_v.at[0])
        pltpu.sync_copy(rows_v.at[0], out_hbm.at[c, pl.ds(s * _RPT + r, _BLK)])


def _dot(a, b):
    return jnp.dot(a, b, preferred_element_type=jnp.float32,
                   precision=lax.Precision.HIGHEST)


def _dinv_of(degp_ref):
    deg = degp_ref[:_N] + degp_ref[_NP:_NP + _N] + 1.0
    return lax.rsqrt(deg)[:, None]


def _tc_matmul1_body(x_ref, w_ref, o_ref):
    o_ref[...] = _dot(x_ref[...], w_ref[...])


_tc_matmul1 = pl.pallas_call(
    _tc_matmul1_body,
    out_shape=jax.ShapeDtypeStruct((_N, _HID), jnp.float32),
)


def _tc_scale_body(h_ref, degp_ref, o_ref):
    o_ref[:_N, :] = _dinv_of(degp_ref) * h_ref[...]
    o_ref[_N:, :] = jnp.zeros((_NP - _N, _HID), jnp.float32)


_tc_scale = pl.pallas_call(
    _tc_scale_body,
    out_shape=jax.ShapeDtypeStruct((_NP, _HID), jnp.float32),
)


def _tc_mid_body(p_ref, g_ref, degp_ref, b_ref, w_ref, o_ref):
    dinv = _dinv_of(degp_ref)
    h = p_ref[0, :_N, :] + p_ref[1, :_N, :] + g_ref[:_N, :]
    h = jnp.maximum(dinv * h + b_ref[...][None, :], 0.0)
    o_ref[:_N, :] = dinv * _dot(h, w_ref[...])
    o_ref[_N:, :] = jnp.zeros((_NP - _N, _HID), jnp.float32)


_tc_mid = pl.pallas_call(
    _tc_mid_body,
    out_shape=jax.ShapeDtypeStruct((_NP, _HID), jnp.float32),
)


def _tc_final_body(p_ref, g_ref, degp_ref, b_ref, batch_ref, lw_ref, lb_ref,
                   o_ref):
    dinv = _dinv_of(degp_ref)
    h = p_ref[0, :_N, :] + p_ref[1, :_N, :] + g_ref[:_N, :]
    h = jnp.maximum(dinv * h + b_ref[...][None, :], 0.0)
    labels = lax.broadcasted_iota(jnp.int32, (1, _NG), 1)
    onehot = (batch_ref[...] == labels).astype(jnp.float32)
    sums = lax.dot_general(onehot, h, (((0,), (0,)), ((), ())),
                           preferred_element_type=jnp.float32,
                           precision=lax.Precision.HIGHEST)
    counts = jnp.sum(onehot, axis=0)[:, None]
    pooled = sums / jnp.maximum(counts, 1.0)
    o_ref[...] = _dot(pooled, lw_ref[...]) + lb_ref[...][None, :]


_tc_final = pl.pallas_call(
    _tc_final_body,
    out_shape=jax.ShapeDtypeStruct((_NG, 2), jnp.float32),
)


def kernel(x, edge_index, batch, W1, b1, W2, b2, W3, b3, lin_W, lin_b):
    pad = _EPAD - _E
    pad_idx = (jnp.arange(pad, dtype=jnp.int32) % (_NP - _N)) + _N
    src = jnp.concatenate([edge_index[0].astype(jnp.int32), pad_idx])
    dst = jnp.concatenate([edge_index[1].astype(jnp.int32), pad_idx])
    src2 = src.reshape(_EPAD // _BLK, _BLK)
    dst2 = dst.reshape(_EPAD // _BLK, _BLK)
    batch2 = batch.astype(jnp.int32).reshape(_N, 1)

    degp = _sc_degree(dst2)
    h1 = _tc_matmul1(x, W1)
    g1 = _tc_scale(h1, degp)
    p1 = _sc_aggregate(g1, src2, dst2)
    g2 = _tc_mid(p1, g1, degp, b1, W2)
    p2 = _sc_aggregate(g2, src2, dst2)
    g3 = _tc_mid(p2, g2, degp, b2, W3)
    p3 = _sc_aggregate(g3, src2, dst2)
    return _tc_final(p3, g3, degp, b3, batch2, lin_W, lin_b)
